# baseline (speedup 1.0000x reference)
import functools
import math

import jax
import jax.numpy as jnp
from jax import lax
from jax.experimental import pallas as pl
from jax.experimental.pallas import tpu as pltpu
from jax.experimental.pallas import tpu_sc as plsc

_N = 10000
_E = 320000
_D = 128
_DH = _D // 2
_G = 100
_INPUT_DIM = 5
_CUTOFF = 10.0
_L = 6

_NS = 16
_EPS = _E // _NS
_CH = 80
_NCHUNK = _EPS // _CH
_NPAD = 10240
_RPS = _NPAD // _NS


def _ssp(x):
    return jnp.maximum(x, 0.0) + jnp.log1p(jnp.exp(-jnp.abs(x))) - math.log(2.0)


def _embed_body(z5_ref, zd_ref, we_ref, be_ref, w1_ref, h_ref, xl_ref):
    h = (jnp.dot(z5_ref[...], we_ref[...], preferred_element_type=jnp.float32)
         + be_ref[...] + zd_ref[...])
    h_ref[...] = h
    xl = jnp.dot(h, w1_ref[...], preferred_element_type=jnp.float32)
    xl_ref[0] = xl[:, :_DH]
    xl_ref[1] = xl[:, _DH:]


def _embed(z5, zd, emblin_W, emblin_b, lin1_W0):
    BN = 1000
    return pl.pallas_call(
        _embed_body,
        grid=(_N // BN,),
        in_specs=[
            pl.BlockSpec((BN, _INPUT_DIM), lambda i: (i, 0)),
            pl.BlockSpec((BN, _D), lambda i: (i, 0)),
            pl.BlockSpec((_INPUT_DIM, _D), lambda i: (0, 0)),
            pl.BlockSpec((1, _D), lambda i: (0, 0)),
            pl.BlockSpec((_D, _D), lambda i: (0, 0)),
        ],
        out_specs=[
            pl.BlockSpec((BN, _D), lambda i: (i, 0)),
            pl.BlockSpec((2, BN, _DH), lambda i: (0, i, 0)),
        ],
        out_shape=[
            jax.ShapeDtypeStruct((_N, _D), jnp.float32),
            jax.ShapeDtypeStruct((2, _N, _DH), jnp.float32),
        ],
    )(z5, zd, emblin_W, emblin_b.reshape(1, _D), lin1_W0)


def _we_body(ea_ref, w1_ref, b1_ref, w2_ref, b2_ref, el_ref, out_ref):
    x = jnp.dot(ea_ref[...], w1_ref[...], preferred_element_type=jnp.float32)
    x = _ssp(x + b1_ref[...])
    we = jnp.dot(x, w2_ref[...], preferred_element_type=jnp.float32) + b2_ref[...]
    el = el_ref[...]
    c = 0.5 * (jnp.cos(el * (math.pi / _CUTOFF)) + 1.0)
    c = jnp.where((el <= _CUTOFF) & (el >= 0.0), c, 0.0)
    we = we * c
    out_ref[0] = we[:, :_DH]
    out_ref[1] = we[:, _DH:]


def _edge_filter(ea, w1, b1, w2, b2, el):
    BE = 1280
    return pl.pallas_call(
        _we_body,
        grid=(_E // BE,),
        in_specs=[
            pl.BlockSpec((BE, _G), lambda i: (i, 0)),
            pl.BlockSpec((_G, _D), lambda i: (0, 0)),
            pl.BlockSpec((1, _D), lambda i: (0, 0)),
            pl.BlockSpec((_D, _D), lambda i: (0, 0)),
            pl.BlockSpec((1, _D), lambda i: (0, 0)),
            pl.BlockSpec((BE, 1), lambda i: (i, 0)),
        ],
        out_specs=pl.BlockSpec((2, BE, _DH), lambda i: (0, i, 0)),
        out_shape=jax.ShapeDtypeStruct((2, _E, _DH), jnp.float32),
    )(ea, w1, b1.reshape(1, _D), w2, b2.reshape(1, _D), el.reshape(_E, 1))


@functools.partial(
    pl.kernel,
    mesh=plsc.VectorSubcoreMesh(core_axis_name="c", subcore_axis_name="s",
                                num_cores=2),
    out_type=jax.ShapeDtypeStruct((2 * _NPAD, _DH), jnp.float32),
    compiler_params=pltpu.CompilerParams(use_tc_tiling_on_sc=False),
    scratch_types=[
        pltpu.VMEM((_NCHUNK, _CH), jnp.int32),
        pltpu.VMEM((_NCHUNK, _CH), jnp.int32),
        pltpu.VMEM((_CH, _DH), jnp.float32),
        pltpu.VMEM((_CH, _DH), jnp.float32),
        pltpu.VMEM((_CH, _DH), jnp.float32),
        pltpu.VMEM((_CH, _DH), jnp.float32),
        pltpu.VMEM_SHARED((_NPAD, _DH), jnp.float32),
        pltpu.SemaphoreType.DMA,
        pltpu.SemaphoreType.DMA,
        pltpu.SemaphoreType.DMA,
        pltpu.SemaphoreType.DMA,
    ],
)
def _cfconv(xl_hbm, we_hbm, src_hbm, dst_hbm, out_hbm,
            src_v, dst_v, rows0, rows1, we0, we1, acc,
            g0, g1, w0, w1):
    cid = lax.axis_index("c")
    sid = lax.axis_index("s")
    arow0 = sid * _RPS
    rows = (rows0, rows1)
    wes = (we0, we1)
    gsem = (g0, g1)
    wsem = (w0, w1)

    def _zbody(i, _):
        rows0[i // 4, pl.ds((i % 4) * 16, 16)] = jnp.zeros((16,), jnp.float32)
        return 0
    lax.fori_loop(0, _CH * 4, _zbody, 0)
    for k in range(_RPS // _CH):
        pltpu.sync_copy(rows0, acc.at[pl.ds(arow0 + k * _CH, _CH), :])
    plsc.subcore_barrier()

    pltpu.sync_copy(src_hbm.at[cid, sid], src_v)
    pltpu.sync_copy(dst_hbm.at[sid], dst_v)

    webase = cid * _E + sid * _EPS

    def _start(c, b):
        pltpu.async_copy(xl_hbm.at[src_v.at[c]], rows[b], gsem[b])
        pltpu.async_copy(we_hbm.at[pl.ds(webase + c * _CH, _CH), :], wes[b],
                         wsem[b])

    def _wait(c, b):
        pltpu.make_async_copy(xl_hbm.at[src_v.at[c]], rows[b], gsem[b]).wait()
        pltpu.make_async_copy(we_hbm.at[pl.ds(webase + c * _CH, _CH), :],
                              wes[b], wsem[b]).wait()

    _start(0, 0)

    def _step(c, b):
        _wait(c, b)

        @pl.when(c + 1 < _NCHUNK)
        def _():
            _start(c + 1, b ^ 1)

        def _mul(r, _):
            for c4 in range(4):
                sl = pl.ds(c4 * 16, 16)
                rows[b][r, sl] = rows[b][r, sl] * wes[b][r, sl]
            return 0
        lax.fori_loop(0, _CH, _mul, 0)

        pltpu.sync_copy(rows[b], acc.at[dst_v.at[c]], add=True)

    def _pair(k, _):
        for b in range(2):
            _step(2 * k + b, b)
        return 0
    lax.fori_loop(0, _NCHUNK // 2, _pair, 0)

    plsc.subcore_barrier()
    for k in range(_RPS // _CH):
        pltpu.sync_copy(acc.at[pl.ds(arow0 + k * _CH, _CH), :], rows0)
        pltpu.sync_copy(
            rows0, out_hbm.at[pl.ds(cid * _NPAD + arow0 + k * _CH, _CH), :])


def _node_body(p_ref, h_ref, w2_ref, b2_ref, wch_ref, wcx_ref, bc_ref,
               w1n_ref, hout_ref, xlout_ref):
    m = jnp.concatenate([p_ref[0], p_ref[1]], axis=1)
    xo = _ssp(jnp.dot(m, w2_ref[...], preferred_element_type=jnp.float32)
              + b2_ref[...])
    upd = (jnp.dot(h_ref[...], wch_ref[...], preferred_element_type=jnp.float32)
           + jnp.dot(xo, wcx_ref[...], preferred_element_type=jnp.float32)
           + bc_ref[...])
    hn = h_ref[...] + upd
    hout_ref[...] = hn
    xl = jnp.dot(hn, w1n_ref[...], preferred_element_type=jnp.float32)
    xlout_ref[0] = xl[:, :_DH]
    xlout_ref[1] = xl[:, _DH:]


def _node_update(p, h, lin2_W, lin2_b, lincat_W, lincat_b, lin1n_W):
    BN = 1000
    p3 = p.reshape(2, _NPAD, _DH)
    return pl.pallas_call(
        _node_body,
        grid=(_N // BN,),
        in_specs=[
            pl.BlockSpec((2, BN, _DH), lambda i: (0, i, 0)),
            pl.BlockSpec((BN, _D), lambda i: (i, 0)),
            pl.BlockSpec((_D, _D), lambda i: (0, 0)),
            pl.BlockSpec((1, _D), lambda i: (0, 0)),
            pl.BlockSpec((_D, _D), lambda i: (0, 0)),
            pl.BlockSpec((_D, _D), lambda i: (0, 0)),
            pl.BlockSpec((1, _D), lambda i: (0, 0)),
            pl.BlockSpec((_D, _D), lambda i: (0, 0)),
        ],
        out_specs=[
            pl.BlockSpec((BN, _D), lambda i: (i, 0)),
            pl.BlockSpec((2, BN, _DH), lambda i: (0, i, 0)),
        ],
        out_shape=[
            jax.ShapeDtypeStruct((_N, _D), jnp.float32),
            jax.ShapeDtypeStruct((2, _N, _DH), jnp.float32),
        ],
    )(p3, h, lin2_W, lin2_b.reshape(1, _D), lincat_W[:_D], lincat_W[_D:],
      lincat_b.reshape(1, _D), lin1n_W)


def kernel(z, edge_index, edge_length, edge_attr, emblin_W, emblin_b,
           mlp1_W, mlp1_b, mlp2_W, mlp2_b, lin1_W, lin2_W, lin2_b,
           lincat_W, lincat_b):
    src = edge_index[0].astype(jnp.int32)
    src2 = jnp.stack([src, src + _N]).reshape(2, _NS, _NCHUNK, _CH)
    dst = edge_index[1].astype(jnp.int32).reshape(_NS, _NCHUNK, _CH)
    h, xl = _embed(z[:, :_INPUT_DIM], z[:, _INPUT_DIM:], emblin_W, emblin_b,
                   lin1_W[0])
    We_list = [_edge_filter(edge_attr, mlp1_W[i], mlp1_b[i], mlp2_W[i],
                            mlp2_b[i], edge_length) for i in range(_L)]
    for i in range(_L):
        p = _cfconv(xl.reshape(2 * _N, _DH), We_list[i].reshape(2 * _E, _DH),
                    src2, dst)
        w1n = lin1_W[(i + 1) % _L]
        h, xl = _node_update(p, h, lin2_W[i], lin2_b[i], lincat_W[i],
                             lincat_b[i], w1n)
    return h


# speedup vs baseline: 1.3257x; 1.3257x over previous
import functools
import math

import jax
import jax.numpy as jnp
from jax import lax
from jax.experimental import pallas as pl
from jax.experimental.pallas import tpu as pltpu
from jax.experimental.pallas import tpu_sc as plsc

_N = 10000
_E = 320000
_D = 128
_G = 100
_INPUT_DIM = 5
_CUTOFF = 10.0
_L = 6

_NW = 32
_EPW = _E // _NW
_CH = 40
_NCHUNK = _EPW // _CH
_NPAD = 10240
_RPS = _NPAD // 16


def _ssp(x):
    return jnp.maximum(x, 0.0) + jnp.log1p(jnp.exp(-jnp.abs(x))) - math.log(2.0)


def _embed_body(z5_ref, zd_ref, we_ref, be_ref, w1_ref, h_ref, xl_ref):
    h = (jnp.dot(z5_ref[...], we_ref[...], preferred_element_type=jnp.float32)
         + be_ref[...] + zd_ref[...])
    h_ref[...] = h
    xl_ref[...] = jnp.dot(h, w1_ref[...], preferred_element_type=jnp.float32)


def _embed(z5, zd, emblin_W, emblin_b, lin1_W0):
    BN = 1000
    return pl.pallas_call(
        _embed_body,
        grid=(_N // BN,),
        in_specs=[
            pl.BlockSpec((BN, _INPUT_DIM), lambda i: (i, 0)),
            pl.BlockSpec((BN, _D), lambda i: (i, 0)),
            pl.BlockSpec((_INPUT_DIM, _D), lambda i: (0, 0)),
            pl.BlockSpec((1, _D), lambda i: (0, 0)),
            pl.BlockSpec((_D, _D), lambda i: (0, 0)),
        ],
        out_specs=[
            pl.BlockSpec((BN, _D), lambda i: (i, 0)),
            pl.BlockSpec((BN, _D), lambda i: (i, 0)),
        ],
        out_shape=[
            jax.ShapeDtypeStruct((_N, _D), jnp.float32),
            jax.ShapeDtypeStruct((_N, _D), jnp.float32),
        ],
    )(z5, zd, emblin_W, emblin_b.reshape(1, _D), lin1_W0)


def _we_body(ea_ref, w1_ref, b1_ref, w2_ref, b2_ref, el_ref, out_ref):
    x = jnp.dot(ea_ref[...], w1_ref[...], preferred_element_type=jnp.float32)
    x = _ssp(x + b1_ref[...])
    we = jnp.dot(x, w2_ref[...], preferred_element_type=jnp.float32) + b2_ref[...]
    el = el_ref[...]
    c = 0.5 * (jnp.cos(el * (math.pi / _CUTOFF)) + 1.0)
    c = jnp.where((el <= _CUTOFF) & (el >= 0.0), c, 0.0)
    out_ref[...] = we * c


def _edge_filter(ea, w1, b1, w2, b2, el):
    BE = 1280
    return pl.pallas_call(
        _we_body,
        grid=(_E // BE,),
        in_specs=[
            pl.BlockSpec((BE, _G), lambda i: (i, 0)),
            pl.BlockSpec((_G, _D), lambda i: (0, 0)),
            pl.BlockSpec((1, _D), lambda i: (0, 0)),
            pl.BlockSpec((_D, _D), lambda i: (0, 0)),
            pl.BlockSpec((1, _D), lambda i: (0, 0)),
            pl.BlockSpec((BE, 1), lambda i: (i, 0)),
        ],
        out_specs=pl.BlockSpec((BE, _D), lambda i: (i, 0)),
        out_shape=jax.ShapeDtypeStruct((_E, _D), jnp.float32),
    )(ea, w1, b1.reshape(1, _D), w2, b2.reshape(1, _D), el.reshape(_E, 1))


@functools.partial(
    pl.kernel,
    mesh=plsc.VectorSubcoreMesh(core_axis_name="c", subcore_axis_name="s",
                                num_cores=2),
    out_type=jax.ShapeDtypeStruct((2 * _NPAD, _D), jnp.float32),
    scratch_types=[
        pltpu.VMEM((_CH,), jnp.int32),
        pltpu.VMEM((_CH,), jnp.int32),
        pltpu.VMEM((_CH,), jnp.int32),
        pltpu.VMEM((_CH,), jnp.int32),
        pltpu.VMEM((_CH, _D), jnp.float32),
        pltpu.VMEM((_CH, _D), jnp.float32),
        pltpu.VMEM((_CH, _D), jnp.float32),
        pltpu.VMEM((_CH, _D), jnp.float32),
        pltpu.VMEM_SHARED((_NPAD, _D), jnp.float32),
        pltpu.SemaphoreType.DMA,
        pltpu.SemaphoreType.DMA,
        pltpu.SemaphoreType.DMA,
        pltpu.SemaphoreType.DMA,
        pltpu.SemaphoreType.DMA,
        pltpu.SemaphoreType.DMA,
    ],
)
def _cfconv(xl_hbm, we_hbm, src_hbm, dst_hbm, out_hbm,
            s0, s1, d0, d1, rows0, rows1, we0, we1, acc,
            i0, i1, g0, g1, w0, w1):
    cid = lax.axis_index("c")
    sid = lax.axis_index("s")
    wid = cid * 16 + sid
    wbase = wid * _EPW
    arow0 = sid * _RPS
    srcs = (s0, s1)
    dsts = (d0, d1)
    rows = (rows0, rows1)
    wes = (we0, we1)
    isem = (i0, i1)
    gsem = (g0, g1)
    wsem = (w0, w1)

    def _zbody(i, _):
        rows0[i // 8, pl.ds((i % 8) * 16, 16)] = jnp.zeros((16,), jnp.float32)
        return 0
    lax.fori_loop(0, _CH * 8, _zbody, 0)
    for k in range(_RPS // _CH):
        pltpu.sync_copy(rows0, acc.at[pl.ds(arow0 + k * _CH, _CH), :])
    plsc.subcore_barrier()

    def _start_idx(c, b):
        base = wbase + c * _CH
        pltpu.async_copy(src_hbm.at[pl.ds(base, _CH)], srcs[b], isem[b])
        pltpu.async_copy(dst_hbm.at[pl.ds(base, _CH)], dsts[b], isem[b])

    def _wait_idx(c, b):
        base = wbase + c * _CH
        pltpu.make_async_copy(src_hbm.at[pl.ds(base, _CH)], srcs[b],
                              isem[b]).wait()
        pltpu.make_async_copy(dst_hbm.at[pl.ds(base, _CH)], dsts[b],
                              isem[b]).wait()

    def _start_data(c, b):
        base = wbase + c * _CH
        pltpu.async_copy(xl_hbm.at[srcs[b]], rows[b], gsem[b])
        pltpu.async_copy(we_hbm.at[pl.ds(base, _CH), :], wes[b], wsem[b])

    def _wait_data(c, b):
        base = wbase + c * _CH
        pltpu.make_async_copy(xl_hbm.at[srcs[b]], rows[b], gsem[b]).wait()
        pltpu.make_async_copy(we_hbm.at[pl.ds(base, _CH), :], wes[b],
                              wsem[b]).wait()

    _start_idx(0, 0)
    _start_idx(1, 1)
    _wait_idx(0, 0)
    _start_data(0, 0)

    def _step(c, b):
        _wait_data(c, b)

        def _mul(r, _):
            for c8 in range(8):
                sl = pl.ds(c8 * 16, 16)
                rows[b][r, sl] = rows[b][r, sl] * wes[b][r, sl]
            return 0
        lax.fori_loop(0, _CH, _mul, 0)

        pltpu.sync_copy(rows[b], acc.at[dsts[b]], add=True)

        @pl.when(c + 2 < _NCHUNK)
        def _():
            _start_idx(c + 2, b)

        @pl.when(c + 1 < _NCHUNK)
        def _():
            _wait_idx(c + 1, b ^ 1)
            _start_data(c + 1, b ^ 1)

    def _pair(k, _):
        for b in range(2):
            _step(2 * k + b, b)
        return 0
    lax.fori_loop(0, _NCHUNK // 2, _pair, 0)

    plsc.subcore_barrier()
    for k in range(_RPS // _CH):
        pltpu.sync_copy(acc.at[pl.ds(arow0 + k * _CH, _CH), :], rows0)
        pltpu.sync_copy(
            rows0, out_hbm.at[pl.ds(cid * _NPAD + arow0 + k * _CH, _CH), :])


def _node_body(p_ref, h_ref, w2_ref, b2_ref, wch_ref, wcx_ref, bc_ref,
               w1n_ref, hout_ref, xlout_ref):
    m = p_ref[0] + p_ref[1]
    xo = _ssp(jnp.dot(m, w2_ref[...], preferred_element_type=jnp.float32)
              + b2_ref[...])
    upd = (jnp.dot(h_ref[...], wch_ref[...], preferred_element_type=jnp.float32)
           + jnp.dot(xo, wcx_ref[...], preferred_element_type=jnp.float32)
           + bc_ref[...])
    hn = h_ref[...] + upd
    hout_ref[...] = hn
    xlout_ref[...] = jnp.dot(hn, w1n_ref[...], preferred_element_type=jnp.float32)


def _node_update(p, h, lin2_W, lin2_b, lincat_W, lincat_b, lin1n_W):
    BN = 1000
    p3 = p.reshape(2, _NPAD, _D)
    return pl.pallas_call(
        _node_body,
        grid=(_N // BN,),
        in_specs=[
            pl.BlockSpec((2, BN, _D), lambda i: (0, i, 0)),
            pl.BlockSpec((BN, _D), lambda i: (i, 0)),
            pl.BlockSpec((_D, _D), lambda i: (0, 0)),
            pl.BlockSpec((1, _D), lambda i: (0, 0)),
            pl.BlockSpec((_D, _D), lambda i: (0, 0)),
            pl.BlockSpec((_D, _D), lambda i: (0, 0)),
            pl.BlockSpec((1, _D), lambda i: (0, 0)),
            pl.BlockSpec((_D, _D), lambda i: (0, 0)),
        ],
        out_specs=[
            pl.BlockSpec((BN, _D), lambda i: (i, 0)),
            pl.BlockSpec((BN, _D), lambda i: (i, 0)),
        ],
        out_shape=[
            jax.ShapeDtypeStruct((_N, _D), jnp.float32),
            jax.ShapeDtypeStruct((_N, _D), jnp.float32),
        ],
    )(p3, h, lin2_W, lin2_b.reshape(1, _D), lincat_W[:_D], lincat_W[_D:],
      lincat_b.reshape(1, _D), lin1n_W)


def kernel(z, edge_index, edge_length, edge_attr, emblin_W, emblin_b,
           mlp1_W, mlp1_b, mlp2_W, mlp2_b, lin1_W, lin2_W, lin2_b,
           lincat_W, lincat_b):
    src = edge_index[0].astype(jnp.int32)
    dst = edge_index[1].astype(jnp.int32)
    h, xl = _embed(z[:, :_INPUT_DIM], z[:, _INPUT_DIM:], emblin_W, emblin_b,
                   lin1_W[0])
    We_list = [_edge_filter(edge_attr, mlp1_W[i], mlp1_b[i], mlp2_W[i],
                            mlp2_b[i], edge_length) for i in range(_L)]
    for i in range(_L):
        p = _cfconv(xl, We_list[i], src, dst)
        w1n = lin1_W[(i + 1) % _L]
        h, xl = _node_update(p, h, lin2_W[i], lin2_b[i], lincat_W[i],
                             lincat_b[i], w1n)
    return h


# speedup vs baseline: 1.3413x; 1.0118x over previous
import functools
import math

import jax
import jax.numpy as jnp
from jax import lax
from jax.experimental import pallas as pl
from jax.experimental.pallas import tpu as pltpu
from jax.experimental.pallas import tpu_sc as plsc

_N = 10000
_E = 320000
_D = 128
_G = 100
_INPUT_DIM = 5
_CUTOFF = 10.0
_L = 6

_NW = 32
_EPW = _E // _NW
_CH = 40
_NCHUNK = _EPW // _CH
_NPAD = 10240
_RPS = _NPAD // 16


def _ssp(x):
    return jnp.maximum(x, 0.0) + jnp.log1p(jnp.exp(-jnp.abs(x))) - math.log(2.0)


def _embed_body(z5_ref, zd_ref, we_ref, be_ref, w1_ref, h_ref, xl_ref):
    h = (jnp.dot(z5_ref[...], we_ref[...], preferred_element_type=jnp.float32)
         + be_ref[...] + zd_ref[...])
    h_ref[...] = h
    xl_ref[...] = jnp.dot(h, w1_ref[...], preferred_element_type=jnp.float32)


def _embed(z5, zd, emblin_W, emblin_b, lin1_W0):
    BN = 1000
    return pl.pallas_call(
        _embed_body,
        grid=(_N // BN,),
        in_specs=[
            pl.BlockSpec((BN, _INPUT_DIM), lambda i: (i, 0)),
            pl.BlockSpec((BN, _D), lambda i: (i, 0)),
            pl.BlockSpec((_INPUT_DIM, _D), lambda i: (0, 0)),
            pl.BlockSpec((1, _D), lambda i: (0, 0)),
            pl.BlockSpec((_D, _D), lambda i: (0, 0)),
        ],
        out_specs=[
            pl.BlockSpec((BN, _D), lambda i: (i, 0)),
            pl.BlockSpec((BN, _D), lambda i: (i, 0)),
        ],
        out_shape=[
            jax.ShapeDtypeStruct((_N, _D), jnp.float32),
            jax.ShapeDtypeStruct((_N, _D), jnp.float32),
        ],
    )(z5, zd, emblin_W, emblin_b.reshape(1, _D), lin1_W0)


def _we_body(ea_ref, w1_ref, b1_ref, w2_ref, b2_ref, el_ref, out_ref):
    x = jnp.dot(ea_ref[...], w1_ref[...], preferred_element_type=jnp.float32)
    x = _ssp(x + b1_ref[...])
    we = jnp.dot(x, w2_ref[...], preferred_element_type=jnp.float32) + b2_ref[...]
    el = el_ref[...]
    c = 0.5 * (jnp.cos(el * (math.pi / _CUTOFF)) + 1.0)
    c = jnp.where((el <= _CUTOFF) & (el >= 0.0), c, 0.0)
    out_ref[...] = we * c


def _edge_filter(ea, w1, b1, w2, b2, el):
    BE = 1280
    return pl.pallas_call(
        _we_body,
        grid=(_E // BE,),
        in_specs=[
            pl.BlockSpec((BE, _G), lambda i: (i, 0)),
            pl.BlockSpec((_G, _D), lambda i: (0, 0)),
            pl.BlockSpec((1, _D), lambda i: (0, 0)),
            pl.BlockSpec((_D, _D), lambda i: (0, 0)),
            pl.BlockSpec((1, _D), lambda i: (0, 0)),
            pl.BlockSpec((BE, 1), lambda i: (i, 0)),
        ],
        out_specs=pl.BlockSpec((BE, _D), lambda i: (i, 0)),
        out_shape=jax.ShapeDtypeStruct((_E, _D), jnp.float32),
    )(ea, w1, b1.reshape(1, _D), w2, b2.reshape(1, _D), el.reshape(_E, 1))


@functools.partial(
    pl.kernel,
    mesh=plsc.VectorSubcoreMesh(core_axis_name="c", subcore_axis_name="s",
                                num_cores=2),
    out_type=jax.ShapeDtypeStruct((2 * _NPAD, _D), jnp.float32),
    scratch_types=[
        pltpu.VMEM((_EPW,), jnp.int32),
        pltpu.VMEM((_CH,), jnp.int32),
        pltpu.VMEM((_CH,), jnp.int32),
        pltpu.VMEM((_CH, _D), jnp.float32),
        pltpu.VMEM((_CH, _D), jnp.float32),
        pltpu.VMEM((_CH, _D), jnp.float32),
        pltpu.VMEM((_CH, _D), jnp.float32),
        pltpu.VMEM_SHARED((_NPAD, _D), jnp.float32),
        pltpu.SemaphoreType.DMA,
        pltpu.SemaphoreType.DMA,
        pltpu.SemaphoreType.DMA,
        pltpu.SemaphoreType.DMA,
        pltpu.SemaphoreType.DMA,
        pltpu.SemaphoreType.DMA,
        pltpu.SemaphoreType.DMA,
        pltpu.SemaphoreType.DMA,
    ],
)
def _cfconv(xl_hbm, we_hbm, src_hbm, dst_hbm, out_hbm,
            src_v, d0, d1, rows0, rows1, we0, we1, acc,
            i0, i1, g0, g1, w0, w1, v0, v1):
    cid = lax.axis_index("c")
    sid = lax.axis_index("s")
    wid = cid * 16 + sid
    wbase = wid * _EPW
    arow0 = sid * _RPS
    dsts = (d0, d1)
    rows = (rows0, rows1)
    wes = (we0, we1)
    isem = (i0, i1)
    gsem = (g0, g1)
    wsem = (w0, w1)
    ssem = (v0, v1)

    def _zbody(i, _):
        rows0[i // 8, pl.ds((i % 8) * 16, 16)] = jnp.zeros((16,), jnp.float32)
        return 0
    lax.fori_loop(0, _CH * 8, _zbody, 0)
    for k in range(_RPS // _CH):
        pltpu.sync_copy(rows0, acc.at[pl.ds(arow0 + k * _CH, _CH), :])
    plsc.subcore_barrier()

    pltpu.sync_copy(src_hbm.at[pl.ds(wbase, _EPW)], src_v)

    def _start_idx(c, b):
        base = wbase + c * _CH
        pltpu.async_copy(dst_hbm.at[pl.ds(base, _CH)], dsts[b], isem[b])

    def _wait_idx(c, b):
        base = wbase + c * _CH
        pltpu.make_async_copy(dst_hbm.at[pl.ds(base, _CH)], dsts[b],
                              isem[b]).wait()

    def _start_data(c, b):
        base = wbase + c * _CH
        pltpu.async_copy(xl_hbm.at[src_v.at[pl.ds(c * _CH, _CH)]], rows[b], gsem[b])
        pltpu.async_copy(we_hbm.at[pl.ds(base, _CH), :], wes[b], wsem[b])

    def _wait_data(c, b):
        base = wbase + c * _CH
        pltpu.make_async_copy(xl_hbm.at[src_v.at[pl.ds(c * _CH, _CH)]], rows[b],
                              gsem[b]).wait()
        pltpu.make_async_copy(we_hbm.at[pl.ds(base, _CH), :], wes[b],
                              wsem[b]).wait()

    def _start_scat(c, b):
        pltpu.async_copy(rows[b], acc.at[dsts[b]], v0 if b == 0 else v1,
                         add=True)

    def _wait_scat(c, b):
        pltpu.make_async_copy(rows[b], acc.at[dsts[b]],
                              v0 if b == 0 else v1).wait()

    _start_idx(0, 0)
    _start_data(0, 0)

    def _step(c, b):
        _wait_data(c, b)

        def _mul(r, _):
            for c8 in range(8):
                sl = pl.ds(c8 * 16, 16)
                rows[b][r, sl] = rows[b][r, sl] * wes[b][r, sl]
            return 0
        lax.fori_loop(0, _CH, _mul, 0)

        _wait_idx(c, b)
        _start_scat(c, b)

        @pl.when(c >= 1)
        def _():
            _wait_scat(c - 1, b ^ 1)

        @pl.when(c + 1 < _NCHUNK)
        def _():
            _start_idx(c + 1, b ^ 1)
            _start_data(c + 1, b ^ 1)

    def _pair(k, _):
        for b in range(2):
            _step(2 * k + b, b)
        return 0
    lax.fori_loop(0, _NCHUNK // 2, _pair, 0)
    _wait_scat(_NCHUNK - 1, 1)

    plsc.subcore_barrier()
    for k in range(_RPS // _CH):
        pltpu.sync_copy(acc.at[pl.ds(arow0 + k * _CH, _CH), :], rows0)
        pltpu.sync_copy(
            rows0, out_hbm.at[pl.ds(cid * _NPAD + arow0 + k * _CH, _CH), :])


def _node_body(p_ref, h_ref, w2_ref, b2_ref, wch_ref, wcx_ref, bc_ref,
               w1n_ref, hout_ref, xlout_ref):
    m = p_ref[0] + p_ref[1]
    xo = _ssp(jnp.dot(m, w2_ref[...], preferred_element_type=jnp.float32)
              + b2_ref[...])
    upd = (jnp.dot(h_ref[...], wch_ref[...], preferred_element_type=jnp.float32)
           + jnp.dot(xo, wcx_ref[...], preferred_element_type=jnp.float32)
           + bc_ref[...])
    hn = h_ref[...] + upd
    hout_ref[...] = hn
    xlout_ref[...] = jnp.dot(hn, w1n_ref[...], preferred_element_type=jnp.float32)


def _node_update(p, h, lin2_W, lin2_b, lincat_W, lincat_b, lin1n_W):
    BN = 1000
    p3 = p.reshape(2, _NPAD, _D)
    return pl.pallas_call(
        _node_body,
        grid=(_N // BN,),
        in_specs=[
            pl.BlockSpec((2, BN, _D), lambda i: (0, i, 0)),
            pl.BlockSpec((BN, _D), lambda i: (i, 0)),
            pl.BlockSpec((_D, _D), lambda i: (0, 0)),
            pl.BlockSpec((1, _D), lambda i: (0, 0)),
            pl.BlockSpec((_D, _D), lambda i: (0, 0)),
            pl.BlockSpec((_D, _D), lambda i: (0, 0)),
            pl.BlockSpec((1, _D), lambda i: (0, 0)),
            pl.BlockSpec((_D, _D), lambda i: (0, 0)),
        ],
        out_specs=[
            pl.BlockSpec((BN, _D), lambda i: (i, 0)),
            pl.BlockSpec((BN, _D), lambda i: (i, 0)),
        ],
        out_shape=[
            jax.ShapeDtypeStruct((_N, _D), jnp.float32),
            jax.ShapeDtypeStruct((_N, _D), jnp.float32),
        ],
    )(p3, h, lin2_W, lin2_b.reshape(1, _D), lincat_W[:_D], lincat_W[_D:],
      lincat_b.reshape(1, _D), lin1n_W)


def kernel(z, edge_index, edge_length, edge_attr, emblin_W, emblin_b,
           mlp1_W, mlp1_b, mlp2_W, mlp2_b, lin1_W, lin2_W, lin2_b,
           lincat_W, lincat_b):
    src = edge_index[0].astype(jnp.int32)
    dst = edge_index[1].astype(jnp.int32)
    h, xl = _embed(z[:, :_INPUT_DIM], z[:, _INPUT_DIM:], emblin_W, emblin_b,
                   lin1_W[0])
    We_list = [_edge_filter(edge_attr, mlp1_W[i], mlp1_b[i], mlp2_W[i],
                            mlp2_b[i], edge_length) for i in range(_L)]
    for i in range(_L):
        p = _cfconv(xl, We_list[i], src, dst)
        w1n = lin1_W[(i + 1) % _L]
        h, xl = _node_update(p, h, lin2_W[i], lin2_b[i], lincat_W[i],
                             lincat_b[i], w1n)
    return h


# speedup vs baseline: 2.2897x; 1.7071x over previous
import functools
import math

import jax
import jax.numpy as jnp
from jax import lax
from jax.experimental import pallas as pl
from jax.experimental.pallas import tpu as pltpu
from jax.experimental.pallas import tpu_sc as plsc

_N = 10000
_E = 320000
_D = 128
_G = 100
_INPUT_DIM = 5
_CUTOFF = 10.0
_L = 6

_NW = 32
_EPW = _E // _NW
_CH = 40
_NCHUNK = _EPW // _CH
_NPAD = 10240
_RPS = _NPAD // 16


def _ssp(x):
    return jnp.maximum(x, 0.0) + jnp.log1p(jnp.exp(-jnp.abs(x))) - math.log(2.0)


def _cutoff_body(el_ref, c_ref):
    el = el_ref[...]
    c = 0.5 * (jnp.cos(el * (math.pi / _CUTOFF)) + 1.0)
    c = jnp.where((el <= _CUTOFF) & (el >= 0.0), c, 0.0)
    c_ref[...] = c.astype(jnp.bfloat16)


def _cutoff(el):
    R = _E // 128
    return pl.pallas_call(
        _cutoff_body,
        grid=(1,),
        in_specs=[pl.BlockSpec((R, 128), lambda i: (0, 0))],
        out_specs=pl.BlockSpec((R, 128), lambda i: (0, 0)),
        out_shape=jax.ShapeDtypeStruct((R, 128), jnp.bfloat16),
    )(el.reshape(R, 128)).reshape(_E, 1)


def _embed_body(z5_ref, zd_ref, we_ref, be_ref, w1_ref, h_ref, xl_ref):
    h = (jnp.dot(z5_ref[...], we_ref[...], preferred_element_type=jnp.float32)
         + be_ref[...] + zd_ref[...])
    h_ref[...] = h
    xl_ref[...] = jnp.dot(h, w1_ref[...], preferred_element_type=jnp.float32)


def _embed(z5, zd, emblin_W, emblin_b, lin1_W0):
    BN = 1000
    return pl.pallas_call(
        _embed_body,
        grid=(_N // BN,),
        in_specs=[
            pl.BlockSpec((BN, _INPUT_DIM), lambda i: (i, 0)),
            pl.BlockSpec((BN, _D), lambda i: (i, 0)),
            pl.BlockSpec((_INPUT_DIM, _D), lambda i: (0, 0)),
            pl.BlockSpec((1, _D), lambda i: (0, 0)),
            pl.BlockSpec((_D, _D), lambda i: (0, 0)),
        ],
        out_specs=[
            pl.BlockSpec((BN, _D), lambda i: (i, 0)),
            pl.BlockSpec((BN, _D), lambda i: (i, 0)),
        ],
        out_shape=[
            jax.ShapeDtypeStruct((_N, _D), jnp.float32),
            jax.ShapeDtypeStruct((_N, _D), jnp.float32),
        ],
    )(z5, zd, emblin_W, emblin_b.reshape(1, _D), lin1_W0)


def _we_body(ea_ref, w1_ref, b1_ref, w2_ref, b2_ref, c_ref, out_ref):
    x = jnp.dot(ea_ref[...], w1_ref[...].astype(jnp.bfloat16),
                preferred_element_type=jnp.float32)
    x = _ssp(x + b1_ref[...])
    we = jnp.dot(x, w2_ref[...], preferred_element_type=jnp.float32) + b2_ref[...]
    out_ref[...] = we * c_ref[...].astype(jnp.float32)


def _edge_filter(ea, w1, b1, w2, b2, c):
    BE = 1280
    return pl.pallas_call(
        _we_body,
        grid=(_E // BE,),
        in_specs=[
            pl.BlockSpec((BE, _G), lambda i: (i, 0)),
            pl.BlockSpec((_G, _D), lambda i: (0, 0)),
            pl.BlockSpec((1, _D), lambda i: (0, 0)),
            pl.BlockSpec((_D, _D), lambda i: (0, 0)),
            pl.BlockSpec((1, _D), lambda i: (0, 0)),
            pl.BlockSpec((BE, 1), lambda i: (i, 0)),
        ],
        out_specs=pl.BlockSpec((BE, _D), lambda i: (i, 0)),
        out_shape=jax.ShapeDtypeStruct((_E, _D), jnp.float32),
    )(ea, w1, b1.reshape(1, _D), w2, b2.reshape(1, _D), c)


@functools.partial(
    pl.kernel,
    mesh=plsc.VectorSubcoreMesh(core_axis_name="c", subcore_axis_name="s",
                                num_cores=2),
    out_type=jax.ShapeDtypeStruct((2 * _NPAD, _D), jnp.float32),
    scratch_types=[
        pltpu.VMEM((_EPW,), jnp.int32),
        pltpu.VMEM((_CH,), jnp.int32),
        pltpu.VMEM((_CH,), jnp.int32),
        pltpu.VMEM((_CH, _D), jnp.float32),
        pltpu.VMEM((_CH, _D), jnp.float32),
        pltpu.VMEM((_CH, _D), jnp.float32),
        pltpu.VMEM((_CH, _D), jnp.float32),
        pltpu.VMEM_SHARED((_NPAD, _D), jnp.float32),
        pltpu.SemaphoreType.DMA,
        pltpu.SemaphoreType.DMA,
        pltpu.SemaphoreType.DMA,
        pltpu.SemaphoreType.DMA,
        pltpu.SemaphoreType.DMA,
        pltpu.SemaphoreType.DMA,
        pltpu.SemaphoreType.DMA,
        pltpu.SemaphoreType.DMA,
    ],
)
def _cfconv(xl_hbm, we_hbm, src_hbm, dst_hbm, out_hbm,
            src_v, d0, d1, rows0, rows1, we0, we1, acc,
            i0, i1, g0, g1, w0, w1, v0, v1):
    cid = lax.axis_index("c")
    sid = lax.axis_index("s")
    wid = cid * 16 + sid
    wbase = wid * _EPW
    arow0 = sid * _RPS
    dsts = (d0, d1)
    rows = (rows0, rows1)
    wes = (we0, we1)
    isem = (i0, i1)
    gsem = (g0, g1)
    wsem = (w0, w1)

    def _zbody(i, _):
        rows0[i // 8, pl.ds((i % 8) * 16, 16)] = jnp.zeros((16,), jnp.float32)
        return 0
    lax.fori_loop(0, _CH * 8, _zbody, 0)
    for k in range(_RPS // _CH):
        pltpu.sync_copy(rows0, acc.at[pl.ds(arow0 + k * _CH, _CH), :])
    plsc.subcore_barrier()

    pltpu.sync_copy(src_hbm.at[pl.ds(wbase, _EPW)], src_v)

    def _start_idx(c, b):
        base = wbase + c * _CH
        pltpu.async_copy(dst_hbm.at[pl.ds(base, _CH)], dsts[b], isem[b])

    def _wait_idx(c, b):
        base = wbase + c * _CH
        pltpu.make_async_copy(dst_hbm.at[pl.ds(base, _CH)], dsts[b],
                              isem[b]).wait()

    def _start_data(c, b):
        base = wbase + c * _CH
        pltpu.async_copy(xl_hbm.at[src_v.at[pl.ds(c * _CH, _CH)]], rows[b],
                         gsem[b])
        pltpu.async_copy(we_hbm.at[pl.ds(base, _CH), :], wes[b], wsem[b])

    def _wait_data(c, b):
        base = wbase + c * _CH
        pltpu.make_async_copy(xl_hbm.at[src_v.at[pl.ds(c * _CH, _CH)]], rows[b],
                              gsem[b]).wait()
        pltpu.make_async_copy(we_hbm.at[pl.ds(base, _CH), :], wes[b],
                              wsem[b]).wait()

    def _start_scat(c, b):
        pltpu.async_copy(rows[b], acc.at[dsts[b]], v0 if b == 0 else v1,
                         add=True)

    def _wait_scat(c, b):
        pltpu.make_async_copy(rows[b], acc.at[dsts[b]],
                              v0 if b == 0 else v1).wait()

    _start_idx(0, 0)
    _start_data(0, 0)

    def _step(c, b):
        _wait_data(c, b)

        def _mul(r, _):
            for c8 in range(8):
                sl = pl.ds(c8 * 16, 16)
                rows[b][r, sl] = rows[b][r, sl] * wes[b][r, sl]
            return 0
        lax.fori_loop(0, _CH, _mul, 0)

        _wait_idx(c, b)
        _start_scat(c, b)

        @pl.when(c >= 1)
        def _():
            _wait_scat(c - 1, b ^ 1)

        @pl.when(c + 1 < _NCHUNK)
        def _():
            _start_idx(c + 1, b ^ 1)
            _start_data(c + 1, b ^ 1)

    def _pair(k, _):
        for b in range(2):
            _step(2 * k + b, b)
        return 0
    lax.fori_loop(0, _NCHUNK // 2, _pair, 0)
    _wait_scat(_NCHUNK - 1, 1)

    plsc.subcore_barrier()
    for k in range(_RPS // _CH):
        pltpu.sync_copy(acc.at[pl.ds(arow0 + k * _CH, _CH), :], rows0)
        pltpu.sync_copy(
            rows0, out_hbm.at[pl.ds(cid * _NPAD + arow0 + k * _CH, _CH), :])


def _node_body(p_ref, h_ref, w2_ref, b2_ref, wch_ref, wcx_ref, bc_ref,
               w1n_ref, hout_ref, xlout_ref):
    m = p_ref[0] + p_ref[1]
    xo = _ssp(jnp.dot(m, w2_ref[...], preferred_element_type=jnp.float32)
              + b2_ref[...])
    upd = (jnp.dot(h_ref[...], wch_ref[...], preferred_element_type=jnp.float32)
           + jnp.dot(xo, wcx_ref[...], preferred_element_type=jnp.float32)
           + bc_ref[...])
    hn = h_ref[...] + upd
    hout_ref[...] = hn
    xlout_ref[...] = jnp.dot(hn, w1n_ref[...], preferred_element_type=jnp.float32)


def _node_update(p, h, lin2_W, lin2_b, lincat_W, lincat_b, lin1n_W):
    BN = 1000
    p3 = p.reshape(2, _NPAD, _D)
    return pl.pallas_call(
        _node_body,
        grid=(_N // BN,),
        in_specs=[
            pl.BlockSpec((2, BN, _D), lambda i: (0, i, 0)),
            pl.BlockSpec((BN, _D), lambda i: (i, 0)),
            pl.BlockSpec((_D, _D), lambda i: (0, 0)),
            pl.BlockSpec((1, _D), lambda i: (0, 0)),
            pl.BlockSpec((_D, _D), lambda i: (0, 0)),
            pl.BlockSpec((_D, _D), lambda i: (0, 0)),
            pl.BlockSpec((1, _D), lambda i: (0, 0)),
            pl.BlockSpec((_D, _D), lambda i: (0, 0)),
        ],
        out_specs=[
            pl.BlockSpec((BN, _D), lambda i: (i, 0)),
            pl.BlockSpec((BN, _D), lambda i: (i, 0)),
        ],
        out_shape=[
            jax.ShapeDtypeStruct((_N, _D), jnp.float32),
            jax.ShapeDtypeStruct((_N, _D), jnp.float32),
        ],
    )(p3, h, lin2_W, lin2_b.reshape(1, _D), lincat_W[:_D], lincat_W[_D:],
      lincat_b.reshape(1, _D), lin1n_W)


def kernel(z, edge_index, edge_length, edge_attr, emblin_W, emblin_b,
           mlp1_W, mlp1_b, mlp2_W, mlp2_b, lin1_W, lin2_W, lin2_b,
           lincat_W, lincat_b):
    src = edge_index[0].astype(jnp.int32)
    dst = edge_index[1].astype(jnp.int32)
    C = _cutoff(edge_length)
    ea16 = edge_attr.astype(jnp.bfloat16)
    h, xl = _embed(z[:, :_INPUT_DIM], z[:, _INPUT_DIM:], emblin_W, emblin_b,
                   lin1_W[0])
    We_list = [_edge_filter(ea16, mlp1_W[i], mlp1_b[i], mlp2_W[i],
                            mlp2_b[i], C) for i in range(_L)]
    for i in range(_L):
        p = _cfconv(xl, We_list[i], src, dst)
        w1n = lin1_W[(i + 1) % _L]
        h, xl = _node_update(p, h, lin2_W[i], lin2_b[i], lincat_W[i],
                             lincat_b[i], w1n)
    return h


# speedup vs baseline: 2.2988x; 1.0040x over previous
import functools
import math

import jax
import jax.numpy as jnp
from jax import lax
from jax.experimental import pallas as pl
from jax.experimental.pallas import tpu as pltpu
from jax.experimental.pallas import tpu_sc as plsc

_N = 10000
_E = 320000
_D = 128
_G = 100
_INPUT_DIM = 5
_CUTOFF = 10.0
_L = 6

_NW = 32
_EPW = _E // _NW
_CH = 40
_NCHUNK = _EPW // _CH
_NPAD = 10240
_RPS = _NPAD // 16


def _ssp(x):
    return jnp.maximum(x, 0.0) + jnp.log1p(jnp.exp(-jnp.abs(x))) - math.log(2.0)


def _cutoff_body(el_ref, c_ref):
    el = el_ref[...]
    c = 0.5 * (jnp.cos(el * (math.pi / _CUTOFF)) + 1.0)
    c = jnp.where((el <= _CUTOFF) & (el >= 0.0), c, 0.0)
    c_ref[...] = c.astype(jnp.bfloat16)


def _cutoff(el):
    R = _E // 128
    return pl.pallas_call(
        _cutoff_body,
        grid=(1,),
        in_specs=[pl.BlockSpec((R, 128), lambda i: (0, 0))],
        out_specs=pl.BlockSpec((R, 128), lambda i: (0, 0)),
        out_shape=jax.ShapeDtypeStruct((R, 128), jnp.bfloat16),
    )(el.reshape(R, 128)).reshape(_E, 1)


def _embed_body(z5_ref, zd_ref, we_ref, be_ref, w1_ref, h_ref, xl_ref):
    h = (jnp.dot(z5_ref[...], we_ref[...], preferred_element_type=jnp.float32)
         + be_ref[...] + zd_ref[...])
    h_ref[...] = h
    xl_ref[...] = jnp.dot(h, w1_ref[...], preferred_element_type=jnp.float32)


def _embed(z5, zd, emblin_W, emblin_b, lin1_W0):
    BN = 1000
    return pl.pallas_call(
        _embed_body,
        grid=(_N // BN,),
        in_specs=[
            pl.BlockSpec((BN, _INPUT_DIM), lambda i: (i, 0)),
            pl.BlockSpec((BN, _D), lambda i: (i, 0)),
            pl.BlockSpec((_INPUT_DIM, _D), lambda i: (0, 0)),
            pl.BlockSpec((1, _D), lambda i: (0, 0)),
            pl.BlockSpec((_D, _D), lambda i: (0, 0)),
        ],
        out_specs=[
            pl.BlockSpec((BN, _D), lambda i: (i, 0)),
            pl.BlockSpec((BN, _D), lambda i: (i, 0)),
        ],
        out_shape=[
            jax.ShapeDtypeStruct((_N, _D), jnp.float32),
            jax.ShapeDtypeStruct((_N, _D), jnp.float32),
        ],
    )(z5, zd, emblin_W, emblin_b.reshape(1, _D), lin1_W0)


def _we_body(ea_ref, w1_ref, b1_ref, w2_ref, b2_ref, c_ref, out_ref):
    x = jnp.dot(ea_ref[...], w1_ref[...].astype(jnp.bfloat16),
                preferred_element_type=jnp.float32)
    x = _ssp(x + b1_ref[...])
    we = jnp.dot(x, w2_ref[...], preferred_element_type=jnp.float32) + b2_ref[...]
    out_ref[...] = we * c_ref[...].astype(jnp.float32)


def _edge_filter(ea, w1, b1, w2, b2, c):
    BE = 1280
    return pl.pallas_call(
        _we_body,
        grid=(_E // BE,),
        in_specs=[
            pl.BlockSpec((BE, _G), lambda i: (i, 0)),
            pl.BlockSpec((_G, _D), lambda i: (0, 0)),
            pl.BlockSpec((1, _D), lambda i: (0, 0)),
            pl.BlockSpec((_D, _D), lambda i: (0, 0)),
            pl.BlockSpec((1, _D), lambda i: (0, 0)),
            pl.BlockSpec((BE, 1), lambda i: (i, 0)),
        ],
        out_specs=pl.BlockSpec((BE, _D), lambda i: (i, 0)),
        out_shape=jax.ShapeDtypeStruct((_E, _D), jnp.float32),
    )(ea, w1, b1.reshape(1, _D), w2, b2.reshape(1, _D), c)


@functools.partial(
    pl.kernel,
    mesh=plsc.VectorSubcoreMesh(core_axis_name="c", subcore_axis_name="s",
                                num_cores=2),
    out_type=jax.ShapeDtypeStruct((2 * _NPAD, _D), jnp.float32),
    scratch_types=[
        pltpu.VMEM((_EPW,), jnp.int32),
        pltpu.VMEM((_CH,), jnp.int32),
        pltpu.VMEM((_CH,), jnp.int32),
        pltpu.VMEM((_CH, _D), jnp.float32),
        pltpu.VMEM((_CH, _D), jnp.float32),
        pltpu.VMEM((_CH, _D), jnp.float32),
        pltpu.VMEM((_CH, _D), jnp.float32),
        pltpu.VMEM_SHARED((_NPAD, _D), jnp.float32),
        pltpu.SemaphoreType.DMA,
        pltpu.SemaphoreType.DMA,
        pltpu.SemaphoreType.DMA,
        pltpu.SemaphoreType.DMA,
        pltpu.SemaphoreType.DMA,
        pltpu.SemaphoreType.DMA,
        pltpu.SemaphoreType.DMA,
        pltpu.SemaphoreType.DMA,
    ],
)
def _cfconv(xl_hbm, we_hbm, src_hbm, dst_hbm, out_hbm,
            src_v, d0, d1, rows0, rows1, we0, we1, acc,
            i0, i1, g0, g1, w0, w1, v0, v1):
    cid = lax.axis_index("c")
    sid = lax.axis_index("s")
    wid = cid * 16 + sid
    wbase = wid * _EPW
    arow0 = sid * _RPS
    dsts = (d0, d1)
    rows = (rows0, rows1)
    wes = (we0, we1)
    isem = (i0, i1)
    gsem = (g0, g1)
    wsem = (w0, w1)

    def _zbody(i, _):
        rows0[i // 8, pl.ds((i % 8) * 16, 16)] = jnp.zeros((16,), jnp.float32)
        return 0
    lax.fori_loop(0, _CH * 8, _zbody, 0)
    for k in range(_RPS // _CH):
        pltpu.sync_copy(rows0, acc.at[pl.ds(arow0 + k * _CH, _CH), :])
    plsc.subcore_barrier()

    pltpu.sync_copy(src_hbm.at[pl.ds(wbase, _EPW)], src_v)

    def _start_idx(c, b):
        base = wbase + c * _CH
        pltpu.async_copy(dst_hbm.at[pl.ds(base, _CH)], dsts[b], isem[b])

    def _wait_idx(c, b):
        base = wbase + c * _CH
        pltpu.make_async_copy(dst_hbm.at[pl.ds(base, _CH)], dsts[b],
                              isem[b]).wait()

    def _start_data(c, b):
        base = wbase + c * _CH
        pltpu.async_copy(xl_hbm.at[src_v.at[pl.ds(c * _CH, _CH)]], rows[b],
                         gsem[b])
        pltpu.async_copy(we_hbm.at[pl.ds(base, _CH), :], wes[b], wsem[b])

    def _wait_data(c, b):
        base = wbase + c * _CH
        pltpu.make_async_copy(xl_hbm.at[src_v.at[pl.ds(c * _CH, _CH)]], rows[b],
                              gsem[b]).wait()
        pltpu.make_async_copy(we_hbm.at[pl.ds(base, _CH), :], wes[b],
                              wsem[b]).wait()

    def _start_scat(c, b):
        pltpu.async_copy(rows[b], acc.at[dsts[b]], v0 if b == 0 else v1,
                         add=True)

    def _wait_scat(c, b):
        pltpu.make_async_copy(rows[b], acc.at[dsts[b]],
                              v0 if b == 0 else v1).wait()

    _start_idx(0, 0)
    _start_data(0, 0)

    def _step(c, b):
        _wait_data(c, b)

        def _mul(r, _):
            for c8 in range(8):
                sl = pl.ds(c8 * 16, 16)
                rows[b][r, sl] = rows[b][r, sl] * wes[b][r, sl]
            return 0
        lax.fori_loop(0, _CH, _mul, 0)

        _wait_idx(c, b)
        _start_scat(c, b)

        @pl.when(c >= 1)
        def _():
            _wait_scat(c - 1, b ^ 1)

        @pl.when(c + 1 < _NCHUNK)
        def _():
            _start_idx(c + 1, b ^ 1)
            _start_data(c + 1, b ^ 1)

    def _pair(k, _):
        for b in range(2):
            _step(2 * k + b, b)
        return 0
    lax.fori_loop(0, _NCHUNK // 2, _pair, 0)
    _wait_scat(_NCHUNK - 1, 1)

    plsc.subcore_barrier()
    pltpu.sync_copy(acc.at[pl.ds(arow0, _RPS), :],
                    out_hbm.at[pl.ds(cid * _NPAD + arow0, _RPS), :])


def _node_body(p_ref, h_ref, w2_ref, b2_ref, wch_ref, wcx_ref, bc_ref,
               w1n_ref, hout_ref, xlout_ref):
    m = p_ref[0] + p_ref[1]
    xo = _ssp(jnp.dot(m, w2_ref[...], preferred_element_type=jnp.float32)
              + b2_ref[...])
    upd = (jnp.dot(h_ref[...], wch_ref[...], preferred_element_type=jnp.float32)
           + jnp.dot(xo, wcx_ref[...], preferred_element_type=jnp.float32)
           + bc_ref[...])
    hn = h_ref[...] + upd
    hout_ref[...] = hn
    xlout_ref[...] = jnp.dot(hn, w1n_ref[...], preferred_element_type=jnp.float32)


def _node_update(p, h, lin2_W, lin2_b, lincat_W, lincat_b, lin1n_W):
    BN = 1000
    p3 = p.reshape(2, _NPAD, _D)
    return pl.pallas_call(
        _node_body,
        grid=(_N // BN,),
        in_specs=[
            pl.BlockSpec((2, BN, _D), lambda i: (0, i, 0)),
            pl.BlockSpec((BN, _D), lambda i: (i, 0)),
            pl.BlockSpec((_D, _D), lambda i: (0, 0)),
            pl.BlockSpec((1, _D), lambda i: (0, 0)),
            pl.BlockSpec((_D, _D), lambda i: (0, 0)),
            pl.BlockSpec((_D, _D), lambda i: (0, 0)),
            pl.BlockSpec((1, _D), lambda i: (0, 0)),
            pl.BlockSpec((_D, _D), lambda i: (0, 0)),
        ],
        out_specs=[
            pl.BlockSpec((BN, _D), lambda i: (i, 0)),
            pl.BlockSpec((BN, _D), lambda i: (i, 0)),
        ],
        out_shape=[
            jax.ShapeDtypeStruct((_N, _D), jnp.float32),
            jax.ShapeDtypeStruct((_N, _D), jnp.float32),
        ],
    )(p3, h, lin2_W, lin2_b.reshape(1, _D), lincat_W[:_D], lincat_W[_D:],
      lincat_b.reshape(1, _D), lin1n_W)


def kernel(z, edge_index, edge_length, edge_attr, emblin_W, emblin_b,
           mlp1_W, mlp1_b, mlp2_W, mlp2_b, lin1_W, lin2_W, lin2_b,
           lincat_W, lincat_b):
    src = edge_index[0].astype(jnp.int32)
    dst = edge_index[1].astype(jnp.int32)
    C = _cutoff(edge_length)
    ea16 = edge_attr.astype(jnp.bfloat16)
    h, xl = _embed(z[:, :_INPUT_DIM], z[:, _INPUT_DIM:], emblin_W, emblin_b,
                   lin1_W[0])
    We_list = [_edge_filter(ea16, mlp1_W[i], mlp1_b[i], mlp2_W[i],
                            mlp2_b[i], C) for i in range(_L)]
    for i in range(_L):
        p = _cfconv(xl, We_list[i], src, dst)
        w1n = lin1_W[(i + 1) % _L]
        h, xl = _node_update(p, h, lin2_W[i], lin2_b[i], lincat_W[i],
                             lincat_b[i], w1n)
    return h
